# Initial kernel scaffold; baseline (speedup 1.0000x reference)
import jax
import jax.numpy as jnp
from jax.experimental import pallas as pl


def kernel(x, edge_index, W_emb, b_emb, W_lin, b_lin, W_conv, b_conv):
    raise NotImplementedError("write your pallas kernel here")


# speedup vs baseline: 100.8072x; 100.8072x over previous
import jax
import jax.numpy as jnp
from jax import lax
from jax.experimental import pallas as pl
from jax.experimental.pallas import tpu as pltpu
from jax.experimental.pallas import tpu_sc as plsc

N = 10000
E = 320000
NP = 10240
NT = 32
NC = 2
ET = E // NT
CH = ET // 16

_MESH = plsc.VectorSubcoreMesh(core_axis_name="c", subcore_axis_name="s")
_SC_PARAMS = pltpu.CompilerParams(needs_layout_passes=False)


def _wid():
    return lax.axis_index("s") * NC + lax.axis_index("c")


def _hist_body(dst_hbm, out_hbm, dst_v, acc_v):
    t = _wid()
    pltpu.sync_copy(dst_hbm.at[pl.ds(t * ET, ET)], dst_v)

    def zero(j, carry):
        acc_v[pl.ds(j * 16, 16)] = jnp.zeros((16,), jnp.float32)
        return carry

    lax.fori_loop(0, NP // 16, zero, 0)

    ones = jnp.ones((16,), jnp.float32)

    def chunk(i, carry):
        d = dst_v[pl.ds(i * 16, 16)]
        plsc.addupdate_scatter(acc_v, [d], ones)
        return carry

    lax.fori_loop(0, CH, chunk, 0)
    pltpu.sync_copy(acc_v, out_hbm.at[t])


_hist = pl.kernel(
    _hist_body,
    out_type=jax.ShapeDtypeStruct((NT, NP), jnp.float32),
    mesh=_MESH,
    compiler_params=_SC_PARAMS,
    scratch_types=[
        pltpu.VMEM((ET,), jnp.int32),
        pltpu.VMEM((NP,), jnp.float32),
    ],
)


def _dense_body(x_ref, we_ref, be_ref, wl_ref, bl_ref, wc_ref, hp_ref, gp_ref):
    hi = lax.Precision.HIGHEST
    m1 = jnp.dot(we_ref[...], wl_ref[...], precision=hi)
    wf = jnp.dot(m1, wc_ref[...], precision=hi)
    b2 = jnp.dot(be_ref[...], wl_ref[...], precision=hi) + bl_ref[...]
    bft = lax.dot_general(wc_ref[...], b2, (((0,), (1,)), ((), ())),
                          precision=hi)
    hct = lax.dot_general(wf, x_ref[...], (((0,), (1,)), ((), ())),
                          precision=hi)
    deg = jnp.sum(hp_ref[...], axis=0, keepdims=True) + 1.0
    dinv = lax.rsqrt(deg)
    g3 = (hct + bft) * dinv[:, :N]
    g3p = jnp.concatenate([g3, jnp.zeros((3, NP - N), jnp.float32)], axis=1)
    gp_ref[...] = jnp.concatenate(
        [g3p, dinv, jnp.zeros((4, NP), jnp.float32)], axis=0)


_dense = pl.pallas_call(
    _dense_body,
    out_shape=jax.ShapeDtypeStruct((8, NP), jnp.float32),
)


def _msg_body(src_hbm, dst_hbm, gp_hbm, out_hbm,
              src_v, dst_v, g0, g1, g2, a0, a1, a2):
    t = _wid()
    pltpu.sync_copy(src_hbm.at[pl.ds(t * ET, ET)], src_v)
    pltpu.sync_copy(dst_hbm.at[pl.ds(t * ET, ET)], dst_v)
    pltpu.sync_copy(gp_hbm.at[0], g0)
    pltpu.sync_copy(gp_hbm.at[1], g1)
    pltpu.sync_copy(gp_hbm.at[2], g2)

    def zero(j, carry):
        z = jnp.zeros((16,), jnp.float32)
        a0[pl.ds(j * 16, 16)] = z
        a1[pl.ds(j * 16, 16)] = z
        a2[pl.ds(j * 16, 16)] = z
        return carry

    lax.fori_loop(0, NP // 16, zero, 0)

    def chunk(i, carry):
        s = src_v[pl.ds(i * 16, 16)]
        d = dst_v[pl.ds(i * 16, 16)]
        v0 = plsc.load_gather(g0, [s])
        v1 = plsc.load_gather(g1, [s])
        v2 = plsc.load_gather(g2, [s])
        plsc.addupdate_scatter(a0, [d], v0)
        plsc.addupdate_scatter(a1, [d], v1)
        plsc.addupdate_scatter(a2, [d], v2)
        return carry

    lax.fori_loop(0, CH, chunk, 0)
    pltpu.sync_copy(a0, out_hbm.at[t])
    pltpu.sync_copy(a1, out_hbm.at[NT + t])
    pltpu.sync_copy(a2, out_hbm.at[2 * NT + t])


_msg = pl.kernel(
    _msg_body,
    out_type=jax.ShapeDtypeStruct((3 * NT, NP), jnp.float32),
    mesh=_MESH,
    compiler_params=_SC_PARAMS,
    scratch_types=[
        pltpu.VMEM((ET,), jnp.int32),
        pltpu.VMEM((ET,), jnp.int32),
        pltpu.VMEM((NP,), jnp.float32),
        pltpu.VMEM((NP,), jnp.float32),
        pltpu.VMEM((NP,), jnp.float32),
        pltpu.VMEM((NP,), jnp.float32),
        pltpu.VMEM((NP,), jnp.float32),
        pltpu.VMEM((NP,), jnp.float32),
    ],
)


def _final_body(sp_ref, gp_ref, bc_ref, out_ref):
    dinv = gp_ref[3:4, :]
    rows = []
    for c in range(3):
        s = jnp.sum(sp_ref[c * NT:(c + 1) * NT, :], axis=0, keepdims=True)
        rows.append(dinv * (s + gp_ref[c:c + 1, :]) + bc_ref[0:1, c:c + 1])
    out_ref[...] = jnp.concatenate(
        rows + [jnp.zeros((5, NP), jnp.float32)], axis=0)


_final = pl.pallas_call(
    _final_body,
    out_shape=jax.ShapeDtypeStruct((8, NP), jnp.float32),
)


def kernel(x, edge_index, W_emb, b_emb, W_lin, b_lin, W_conv, b_conv):
    src = edge_index[0]
    dst = edge_index[1]
    hp = _hist(dst)
    gp = _dense(x, W_emb, b_emb.reshape(1, -1), W_lin, b_lin.reshape(1, -1),
                W_conv, hp)
    sp = _msg(src, dst, gp)
    out_t = _final(sp, gp, b_conv.reshape(1, -1))
    return out_t[:3, :N].T


# speedup vs baseline: 105.2537x; 1.0441x over previous
import jax
import jax.numpy as jnp
from jax import lax
from jax.experimental import pallas as pl
from jax.experimental.pallas import tpu as pltpu
from jax.experimental.pallas import tpu_sc as plsc

N = 10000
E = 320000
NP = 10240
NT = 32
NC = 2
ET = E // NT
CH = ET // 16

_MESH = plsc.VectorSubcoreMesh(core_axis_name="c", subcore_axis_name="s")
_SC_PARAMS = pltpu.CompilerParams(needs_layout_passes=False)


def _wid():
    return lax.axis_index("s") * NC + lax.axis_index("c")


def _hist_body(dst_hbm, out_hbm, dst_v, acc_v):
    t = _wid()
    pltpu.sync_copy(dst_hbm.at[pl.ds(t * ET, ET)], dst_v)

    def zero(j, carry):
        base = j * 256
        for u in range(16):
            acc_v[pl.ds(base + u * 16, 16)] = jnp.zeros((16,), jnp.float32)
        return carry

    lax.fori_loop(0, NP // 256, zero, 0)

    ones = jnp.ones((16,), jnp.float32)
    U = 25

    def chunk(i, carry):
        base = i * (16 * U)
        for u in range(U):
            d = dst_v[pl.ds(base + u * 16, 16)]
            plsc.addupdate_scatter(acc_v, [d], ones)
        return carry

    lax.fori_loop(0, CH // U, chunk, 0)
    pltpu.sync_copy(acc_v, out_hbm.at[t])


_hist = pl.kernel(
    _hist_body,
    out_type=jax.ShapeDtypeStruct((NT, NP), jnp.float32),
    mesh=_MESH,
    compiler_params=_SC_PARAMS,
    scratch_types=[
        pltpu.VMEM((ET,), jnp.int32),
        pltpu.VMEM((NP,), jnp.float32),
    ],
)


def _dense_body(x_ref, we_ref, be_ref, wl_ref, bl_ref, wc_ref, hp_ref, gp_ref):
    hi = lax.Precision.HIGHEST
    m1 = jnp.dot(we_ref[...], wl_ref[...], precision=hi)
    wf = jnp.dot(m1, wc_ref[...], precision=hi)
    b2 = jnp.dot(be_ref[...], wl_ref[...], precision=hi) + bl_ref[...]
    bft = lax.dot_general(wc_ref[...], b2, (((0,), (1,)), ((), ())),
                          precision=hi)
    hct = lax.dot_general(wf, x_ref[...], (((0,), (1,)), ((), ())),
                          precision=hi)
    deg = jnp.sum(hp_ref[...], axis=0, keepdims=True) + 1.0
    dinv = lax.rsqrt(deg)
    g3 = (hct + bft) * dinv[:, :N]
    g3p = jnp.concatenate([g3, jnp.zeros((3, NP - N), jnp.float32)], axis=1)
    gp_ref[...] = jnp.concatenate(
        [g3p, dinv, jnp.zeros((4, NP), jnp.float32)], axis=0)


_dense = pl.pallas_call(
    _dense_body,
    out_shape=jax.ShapeDtypeStruct((8, NP), jnp.float32),
)


def _msg_body(src_hbm, dst_hbm, gp_hbm, out_hbm,
              src_v, dst_v, g0, g1, g2, a0, a1, a2):
    t = _wid()
    pltpu.sync_copy(src_hbm.at[pl.ds(t * ET, ET)], src_v)
    pltpu.sync_copy(dst_hbm.at[pl.ds(t * ET, ET)], dst_v)
    pltpu.sync_copy(gp_hbm.at[0], g0)
    pltpu.sync_copy(gp_hbm.at[1], g1)
    pltpu.sync_copy(gp_hbm.at[2], g2)

    def zero(j, carry):
        base = j * 128
        for u in range(8):
            z = jnp.zeros((16,), jnp.float32)
            a0[pl.ds(base + u * 16, 16)] = z
            a1[pl.ds(base + u * 16, 16)] = z
            a2[pl.ds(base + u * 16, 16)] = z
        return carry

    lax.fori_loop(0, NP // 128, zero, 0)

    U = 25

    def chunk(i, carry):
        base = i * (16 * U)
        for u in range(U):
            s = src_v[pl.ds(base + u * 16, 16)]
            d = dst_v[pl.ds(base + u * 16, 16)]
            v0 = plsc.load_gather(g0, [s])
            v1 = plsc.load_gather(g1, [s])
            v2 = plsc.load_gather(g2, [s])
            plsc.addupdate_scatter(a0, [d], v0)
            plsc.addupdate_scatter(a1, [d], v1)
            plsc.addupdate_scatter(a2, [d], v2)
        return carry

    lax.fori_loop(0, CH // U, chunk, 0)
    pltpu.sync_copy(a0, out_hbm.at[t])
    pltpu.sync_copy(a1, out_hbm.at[NT + t])
    pltpu.sync_copy(a2, out_hbm.at[2 * NT + t])


_msg = pl.kernel(
    _msg_body,
    out_type=jax.ShapeDtypeStruct((3 * NT, NP), jnp.float32),
    mesh=_MESH,
    compiler_params=_SC_PARAMS,
    scratch_types=[
        pltpu.VMEM((ET,), jnp.int32),
        pltpu.VMEM((ET,), jnp.int32),
        pltpu.VMEM((NP,), jnp.float32),
        pltpu.VMEM((NP,), jnp.float32),
        pltpu.VMEM((NP,), jnp.float32),
        pltpu.VMEM((NP,), jnp.float32),
        pltpu.VMEM((NP,), jnp.float32),
        pltpu.VMEM((NP,), jnp.float32),
    ],
)


def _final_body(sp_ref, gp_ref, bc_ref, out_ref):
    dinv = gp_ref[3:4, :]
    rows = []
    for c in range(3):
        s = jnp.sum(sp_ref[c * NT:(c + 1) * NT, :], axis=0, keepdims=True)
        rows.append(dinv * (s + gp_ref[c:c + 1, :]) + bc_ref[0:1, c:c + 1])
    out_ref[...] = jnp.concatenate(
        rows + [jnp.zeros((5, NP), jnp.float32)], axis=0)


_final = pl.pallas_call(
    _final_body,
    out_shape=jax.ShapeDtypeStruct((8, NP), jnp.float32),
)


def kernel(x, edge_index, W_emb, b_emb, W_lin, b_lin, W_conv, b_conv):
    src = edge_index[0]
    dst = edge_index[1]
    hp = _hist(dst)
    gp = _dense(x, W_emb, b_emb.reshape(1, -1), W_lin, b_lin.reshape(1, -1),
                W_conv, hp)
    sp = _msg(src, dst, gp)
    out_t = _final(sp, gp, b_conv.reshape(1, -1))
    return out_t[:3, :N].T


# speedup vs baseline: 109.7912x; 1.0431x over previous
import jax
import jax.numpy as jnp
from jax import lax
from jax.experimental import pallas as pl
from jax.experimental.pallas import tpu as pltpu
from jax.experimental.pallas import tpu_sc as plsc

N = 10000
E = 320000
NP = 10240
NT = 32
NC = 2
ET = E // NT
CH = ET // 16

_MESH = plsc.VectorSubcoreMesh(core_axis_name="c", subcore_axis_name="s")
_SC_PARAMS = pltpu.CompilerParams(needs_layout_passes=False)


def _wid():
    return lax.axis_index("s") * NC + lax.axis_index("c")


def _hist_body(dst_hbm, out_hbm, dst_v, acc_v):
    t = _wid()
    pltpu.sync_copy(dst_hbm.at[pl.ds(t * ET, ET)], dst_v)

    def zero(j, carry):
        base = j * 256
        for u in range(16):
            acc_v[pl.ds(base + u * 16, 16)] = jnp.zeros((16,), jnp.float32)
        return carry

    lax.fori_loop(0, NP // 256, zero, 0)

    ones = jnp.ones((16,), jnp.float32)
    U = 25

    def chunk(i, carry):
        base = i * (16 * U)
        for u in range(U):
            d = dst_v[pl.ds(base + u * 16, 16)]
            plsc.addupdate_scatter(acc_v, [d], ones)
        return carry

    lax.fori_loop(0, CH // U, chunk, 0)
    pltpu.sync_copy(acc_v, out_hbm.at[t])


_hist = pl.kernel(
    _hist_body,
    out_type=jax.ShapeDtypeStruct((NT, NP), jnp.float32),
    mesh=_MESH,
    compiler_params=_SC_PARAMS,
    scratch_types=[
        pltpu.VMEM((ET,), jnp.int32),
        pltpu.VMEM((NP,), jnp.float32),
    ],
)


def _dense_body(x_ref, we_ref, be_ref, wl_ref, bl_ref, wc_ref, hc_ref):
    hi = lax.Precision.HIGHEST
    m1 = jnp.dot(we_ref[...], wl_ref[...], precision=hi)
    wf = jnp.dot(m1, wc_ref[...], precision=hi)
    b2 = jnp.dot(be_ref[...], wl_ref[...], precision=hi) + bl_ref[...]
    bft = lax.dot_general(wc_ref[...], b2, (((0,), (1,)), ((), ())),
                          precision=hi)
    hct = lax.dot_general(wf, x_ref[...], (((0,), (1,)), ((), ())),
                          precision=hi) + bft
    hc_ref[...] = jnp.concatenate(
        [hct, jnp.zeros((3, NP - N), jnp.float32)], axis=1)


_dense = pl.pallas_call(
    _dense_body,
    out_shape=jax.ShapeDtypeStruct((3, NP), jnp.float32),
)


def _scale_body(hc_ref, hp_ref, gp_ref):
    deg = jnp.sum(hp_ref[...], axis=0, keepdims=True) + 1.0
    dinv = lax.rsqrt(deg)
    g3 = hc_ref[...] * dinv
    gp_ref[...] = jnp.concatenate(
        [g3, dinv, jnp.zeros((4, NP), jnp.float32)], axis=0)


_scale = pl.pallas_call(
    _scale_body,
    out_shape=jax.ShapeDtypeStruct((8, NP), jnp.float32),
)


def _msg_body(src_hbm, dst_hbm, gp_hbm, out_hbm,
              src_v, dst_v, g0, g1, g2, a0, a1, a2):
    t = _wid()
    pltpu.sync_copy(src_hbm.at[pl.ds(t * ET, ET)], src_v)
    pltpu.sync_copy(dst_hbm.at[pl.ds(t * ET, ET)], dst_v)
    pltpu.sync_copy(gp_hbm.at[0], g0)
    pltpu.sync_copy(gp_hbm.at[1], g1)
    pltpu.sync_copy(gp_hbm.at[2], g2)

    def zero(j, carry):
        base = j * 128
        for u in range(8):
            z = jnp.zeros((16,), jnp.float32)
            a0[pl.ds(base + u * 16, 16)] = z
            a1[pl.ds(base + u * 16, 16)] = z
            a2[pl.ds(base + u * 16, 16)] = z
        return carry

    lax.fori_loop(0, NP // 128, zero, 0)

    U = 25

    def chunk(i, carry):
        base = i * (16 * U)
        for u in range(U):
            s = src_v[pl.ds(base + u * 16, 16)]
            d = dst_v[pl.ds(base + u * 16, 16)]
            v0 = plsc.load_gather(g0, [s])
            v1 = plsc.load_gather(g1, [s])
            v2 = plsc.load_gather(g2, [s])
            plsc.addupdate_scatter(a0, [d], v0)
            plsc.addupdate_scatter(a1, [d], v1)
            plsc.addupdate_scatter(a2, [d], v2)
        return carry

    lax.fori_loop(0, CH // U, chunk, 0)
    pltpu.sync_copy(a0, out_hbm.at[t])
    pltpu.sync_copy(a1, out_hbm.at[NT + t])
    pltpu.sync_copy(a2, out_hbm.at[2 * NT + t])


_msg = pl.kernel(
    _msg_body,
    out_type=jax.ShapeDtypeStruct((3 * NT, NP), jnp.float32),
    mesh=_MESH,
    compiler_params=_SC_PARAMS,
    scratch_types=[
        pltpu.VMEM((ET,), jnp.int32),
        pltpu.VMEM((ET,), jnp.int32),
        pltpu.VMEM((NP,), jnp.float32),
        pltpu.VMEM((NP,), jnp.float32),
        pltpu.VMEM((NP,), jnp.float32),
        pltpu.VMEM((NP,), jnp.float32),
        pltpu.VMEM((NP,), jnp.float32),
        pltpu.VMEM((NP,), jnp.float32),
    ],
)


def _final_body(sp_ref, gp_ref, bc_ref, out_ref):
    dinv = gp_ref[3:4, :]
    rows = []
    for c in range(3):
        s = jnp.sum(sp_ref[c * NT:(c + 1) * NT, :], axis=0, keepdims=True)
        rows.append(dinv * (s + gp_ref[c:c + 1, :]) + bc_ref[0:1, c:c + 1])
    out_ref[...] = jnp.concatenate(
        rows + [jnp.zeros((5, NP), jnp.float32)], axis=0)


_final = pl.pallas_call(
    _final_body,
    out_shape=jax.ShapeDtypeStruct((8, NP), jnp.float32),
)


def kernel(x, edge_index, W_emb, b_emb, W_lin, b_lin, W_conv, b_conv):
    src = edge_index[0]
    dst = edge_index[1]
    hp = _hist(dst)
    hc = _dense(x, W_emb, b_emb.reshape(1, -1), W_lin, b_lin.reshape(1, -1),
                W_conv)
    gp = _scale(hc, hp)
    sp = _msg(src, dst, gp)
    out_t = _final(sp, gp, b_conv.reshape(1, -1))
    return out_t[:3, :N].T


# speedup vs baseline: 114.6424x; 1.0442x over previous
import jax
import jax.numpy as jnp
from jax import lax
from jax.experimental import pallas as pl
from jax.experimental.pallas import tpu as pltpu
from jax.experimental.pallas import tpu_sc as plsc

N = 10000
E = 320000
NP = 10240
NT = 32
NC = 2
ET = E // NT
CH = ET // 16

_MESH = plsc.VectorSubcoreMesh(core_axis_name="c", subcore_axis_name="s")
_SC_PARAMS = pltpu.CompilerParams(needs_layout_passes=False)


def _wid():
    return lax.axis_index("s") * NC + lax.axis_index("c")


def _hist_body(dst_hbm, out_hbm, dst_v, acc_v):
    t = _wid()
    pltpu.sync_copy(dst_hbm.at[pl.ds(t * ET, ET)], dst_v)

    def zero(j, carry):
        base = j * 256
        for u in range(16):
            acc_v[pl.ds(base + u * 16, 16)] = jnp.zeros((16,), jnp.float32)
        return carry

    lax.fori_loop(0, NP // 256, zero, 0)

    ones = jnp.ones((16,), jnp.float32)

    @plsc.parallel_loop(0, CH, unroll=8)
    def _(i):
        d = dst_v[pl.ds(i * 16, 16)]
        plsc.addupdate_scatter(acc_v, [d], ones)
    pltpu.sync_copy(acc_v, out_hbm.at[t])


_hist = pl.kernel(
    _hist_body,
    out_type=jax.ShapeDtypeStruct((NT, NP), jnp.float32),
    mesh=_MESH,
    compiler_params=_SC_PARAMS,
    scratch_types=[
        pltpu.VMEM((ET,), jnp.int32),
        pltpu.VMEM((NP,), jnp.float32),
    ],
)


def _dense_body(x_ref, we_ref, be_ref, wl_ref, bl_ref, wc_ref, hc_ref):
    hi = lax.Precision.HIGHEST
    m1 = jnp.dot(we_ref[...], wl_ref[...], precision=hi)
    wf = jnp.dot(m1, wc_ref[...], precision=hi)
    b2 = jnp.dot(be_ref[...], wl_ref[...], precision=hi) + bl_ref[...]
    bft = lax.dot_general(wc_ref[...], b2, (((0,), (1,)), ((), ())),
                          precision=hi)
    hct = lax.dot_general(wf, x_ref[...], (((0,), (1,)), ((), ())),
                          precision=hi) + bft
    hc_ref[...] = jnp.concatenate(
        [hct, jnp.zeros((3, NP - N), jnp.float32)], axis=1)


_dense = pl.pallas_call(
    _dense_body,
    out_shape=jax.ShapeDtypeStruct((3, NP), jnp.float32),
)


def _scale_body(hc_ref, hp_ref, gp_ref):
    deg = jnp.sum(hp_ref[...], axis=0, keepdims=True) + 1.0
    dinv = lax.rsqrt(deg)
    g3 = hc_ref[...] * dinv
    gp_ref[...] = jnp.concatenate(
        [g3, dinv, jnp.zeros((4, NP), jnp.float32)], axis=0)


_scale = pl.pallas_call(
    _scale_body,
    out_shape=jax.ShapeDtypeStruct((8, NP), jnp.float32),
)


def _msg_body(src_hbm, dst_hbm, gp_hbm, out_hbm,
              src_v, dst_v, g0, g1, g2, a0, a1, a2):
    t = _wid()
    pltpu.sync_copy(src_hbm.at[pl.ds(t * ET, ET)], src_v)
    pltpu.sync_copy(dst_hbm.at[pl.ds(t * ET, ET)], dst_v)
    pltpu.sync_copy(gp_hbm.at[0], g0)
    pltpu.sync_copy(gp_hbm.at[1], g1)
    pltpu.sync_copy(gp_hbm.at[2], g2)

    def zero(j, carry):
        base = j * 128
        for u in range(8):
            z = jnp.zeros((16,), jnp.float32)
            a0[pl.ds(base + u * 16, 16)] = z
            a1[pl.ds(base + u * 16, 16)] = z
            a2[pl.ds(base + u * 16, 16)] = z
        return carry

    lax.fori_loop(0, NP // 128, zero, 0)

    @plsc.parallel_loop(0, CH, unroll=8)
    def _(i):
        s = src_v[pl.ds(i * 16, 16)]
        d = dst_v[pl.ds(i * 16, 16)]
        v0 = plsc.load_gather(g0, [s])
        v1 = plsc.load_gather(g1, [s])
        v2 = plsc.load_gather(g2, [s])
        plsc.addupdate_scatter(a0, [d], v0)
        plsc.addupdate_scatter(a1, [d], v1)
        plsc.addupdate_scatter(a2, [d], v2)
    pltpu.sync_copy(a0, out_hbm.at[t])
    pltpu.sync_copy(a1, out_hbm.at[NT + t])
    pltpu.sync_copy(a2, out_hbm.at[2 * NT + t])


_msg = pl.kernel(
    _msg_body,
    out_type=jax.ShapeDtypeStruct((3 * NT, NP), jnp.float32),
    mesh=_MESH,
    compiler_params=_SC_PARAMS,
    scratch_types=[
        pltpu.VMEM((ET,), jnp.int32),
        pltpu.VMEM((ET,), jnp.int32),
        pltpu.VMEM((NP,), jnp.float32),
        pltpu.VMEM((NP,), jnp.float32),
        pltpu.VMEM((NP,), jnp.float32),
        pltpu.VMEM((NP,), jnp.float32),
        pltpu.VMEM((NP,), jnp.float32),
        pltpu.VMEM((NP,), jnp.float32),
    ],
)


def _final_body(sp_ref, gp_ref, bc_ref, out_ref):
    dinv = gp_ref[3:4, :]
    rows = []
    for c in range(3):
        s = jnp.sum(sp_ref[c * NT:(c + 1) * NT, :], axis=0, keepdims=True)
        rows.append(dinv * (s + gp_ref[c:c + 1, :]) + bc_ref[0:1, c:c + 1])
    out_ref[...] = jnp.concatenate(
        rows + [jnp.zeros((5, NP), jnp.float32)], axis=0)


_final = pl.pallas_call(
    _final_body,
    out_shape=jax.ShapeDtypeStruct((8, NP), jnp.float32),
)


def kernel(x, edge_index, W_emb, b_emb, W_lin, b_lin, W_conv, b_conv):
    src = edge_index[0]
    dst = edge_index[1]
    hp = _hist(dst)
    hc = _dense(x, W_emb, b_emb.reshape(1, -1), W_lin, b_lin.reshape(1, -1),
                W_conv)
    gp = _scale(hc, hp)
    sp = _msg(src, dst, gp)
    out_t = _final(sp, gp, b_conv.reshape(1, -1))
    return out_t[:3, :N].T


# speedup vs baseline: 117.8667x; 1.0281x over previous
import jax
import jax.numpy as jnp
from jax import lax
from jax.experimental import pallas as pl
from jax.experimental.pallas import tpu as pltpu
from jax.experimental.pallas import tpu_sc as plsc

N = 10000
E = 320000
NP = 10240
NT = 32
NC = 2
ET = E // NT
CH = ET // 16

_MESH = plsc.VectorSubcoreMesh(core_axis_name="c", subcore_axis_name="s")
_SC_PARAMS = pltpu.CompilerParams(needs_layout_passes=False)


def _wid():
    return lax.axis_index("s") * NC + lax.axis_index("c")


def _hist_body(dst_hbm, out_hbm, dst_v, acc_v, sem):
    t = _wid()
    cp = pltpu.async_copy(dst_hbm.at[pl.ds(t * ET, ET)], dst_v, sem)

    @plsc.parallel_loop(0, NP // 256, unroll=4)
    def _(j):
        base = j * 256
        for u in range(16):
            acc_v[pl.ds(base + u * 16, 16)] = jnp.zeros((16,), jnp.float32)

    cp.wait()
    ones = jnp.ones((16,), jnp.float32)

    @plsc.parallel_loop(0, CH, unroll=16)
    def _(i):
        d = dst_v[pl.ds(i * 16, 16)]
        plsc.addupdate_scatter(acc_v, [d], ones)
    pltpu.sync_copy(acc_v, out_hbm.at[t])


_hist = pl.kernel(
    _hist_body,
    out_type=jax.ShapeDtypeStruct((NT, NP), jnp.float32),
    mesh=_MESH,
    compiler_params=_SC_PARAMS,
    scratch_types=[
        pltpu.VMEM((ET,), jnp.int32),
        pltpu.VMEM((NP,), jnp.float32),
        pltpu.SemaphoreType.DMA,
    ],
)


def _dense_body(x_ref, we_ref, be_ref, wl_ref, bl_ref, wc_ref, hc_ref):
    hi = lax.Precision.HIGHEST
    m1 = jnp.dot(we_ref[...], wl_ref[...], precision=hi)
    wf = jnp.dot(m1, wc_ref[...], precision=hi)
    b2 = jnp.dot(be_ref[...], wl_ref[...], precision=hi) + bl_ref[...]
    bft = lax.dot_general(wc_ref[...], b2, (((0,), (1,)), ((), ())),
                          precision=hi)
    hct = lax.dot_general(wf, x_ref[...], (((0,), (1,)), ((), ())),
                          precision=hi) + bft
    hc_ref[...] = jnp.concatenate(
        [hct, jnp.zeros((3, NP - N), jnp.float32)], axis=1)


_dense = pl.pallas_call(
    _dense_body,
    out_shape=jax.ShapeDtypeStruct((3, NP), jnp.float32),
)


def _scale_body(hc_ref, hp_ref, gp_ref):
    deg = jnp.sum(hp_ref[...], axis=0, keepdims=True) + 1.0
    dinv = lax.rsqrt(deg)
    g3 = hc_ref[...] * dinv
    gp_ref[...] = jnp.concatenate(
        [g3, dinv, jnp.zeros((4, NP), jnp.float32)], axis=0)


_scale = pl.pallas_call(
    _scale_body,
    out_shape=jax.ShapeDtypeStruct((8, NP), jnp.float32),
)


def _msg_body(src_hbm, dst_hbm, gp_hbm, out_hbm,
              src_v, dst_v, g0, g1, g2, a0, a1, a2, sem):
    t = _wid()
    cps = [
        pltpu.async_copy(src_hbm.at[pl.ds(t * ET, ET)], src_v, sem),
        pltpu.async_copy(dst_hbm.at[pl.ds(t * ET, ET)], dst_v, sem),
        pltpu.async_copy(gp_hbm.at[0], g0, sem),
        pltpu.async_copy(gp_hbm.at[1], g1, sem),
        pltpu.async_copy(gp_hbm.at[2], g2, sem),
    ]

    @plsc.parallel_loop(0, NP // 128, unroll=4)
    def _(j):
        base = j * 128
        for u in range(8):
            z = jnp.zeros((16,), jnp.float32)
            a0[pl.ds(base + u * 16, 16)] = z
            a1[pl.ds(base + u * 16, 16)] = z
            a2[pl.ds(base + u * 16, 16)] = z

    for cp in cps:
        cp.wait()

    @plsc.parallel_loop(0, CH, unroll=16)
    def _(i):
        s = src_v[pl.ds(i * 16, 16)]
        d = dst_v[pl.ds(i * 16, 16)]
        v0 = plsc.load_gather(g0, [s])
        v1 = plsc.load_gather(g1, [s])
        v2 = plsc.load_gather(g2, [s])
        plsc.addupdate_scatter(a0, [d], v0)
        plsc.addupdate_scatter(a1, [d], v1)
        plsc.addupdate_scatter(a2, [d], v2)
    pltpu.sync_copy(a0, out_hbm.at[t])
    pltpu.sync_copy(a1, out_hbm.at[NT + t])
    pltpu.sync_copy(a2, out_hbm.at[2 * NT + t])


_msg = pl.kernel(
    _msg_body,
    out_type=jax.ShapeDtypeStruct((3 * NT, NP), jnp.float32),
    mesh=_MESH,
    compiler_params=_SC_PARAMS,
    scratch_types=[
        pltpu.VMEM((ET,), jnp.int32),
        pltpu.VMEM((ET,), jnp.int32),
        pltpu.VMEM((NP,), jnp.float32),
        pltpu.VMEM((NP,), jnp.float32),
        pltpu.VMEM((NP,), jnp.float32),
        pltpu.VMEM((NP,), jnp.float32),
        pltpu.VMEM((NP,), jnp.float32),
        pltpu.VMEM((NP,), jnp.float32),
        pltpu.SemaphoreType.DMA,
    ],
)


def _final_body(sp_ref, gp_ref, bc_ref, out_ref):
    dinv = gp_ref[3:4, :]
    rows = []
    for c in range(3):
        s = jnp.sum(sp_ref[c * NT:(c + 1) * NT, :], axis=0, keepdims=True)
        rows.append(dinv * (s + gp_ref[c:c + 1, :]) + bc_ref[0:1, c:c + 1])
    out_ref[...] = jnp.concatenate(
        rows + [jnp.zeros((5, NP), jnp.float32)], axis=0)


_final = pl.pallas_call(
    _final_body,
    out_shape=jax.ShapeDtypeStruct((8, NP), jnp.float32),
)


def kernel(x, edge_index, W_emb, b_emb, W_lin, b_lin, W_conv, b_conv):
    src = edge_index[0]
    dst = edge_index[1]
    hp = _hist(dst)
    hc = _dense(x, W_emb, b_emb.reshape(1, -1), W_lin, b_lin.reshape(1, -1),
                W_conv)
    gp = _scale(hc, hp)
    sp = _msg(src, dst, gp)
    out_t = _final(sp, gp, b_conv.reshape(1, -1))
    return out_t[:3, :N].T


# speedup vs baseline: 118.0803x; 1.0018x over previous
import jax
import jax.numpy as jnp
from jax import lax
from jax.experimental import pallas as pl
from jax.experimental.pallas import tpu as pltpu
from jax.experimental.pallas import tpu_sc as plsc

N = 10000
E = 320000
NP = 10240
NT = 32
NC = 2
ET = E // NT
CH = ET // 16

_MESH = plsc.VectorSubcoreMesh(core_axis_name="c", subcore_axis_name="s")
_SC_PARAMS = pltpu.CompilerParams(needs_layout_passes=False, disable_bounds_checks=True, disable_semaphore_checks=True)


def _wid():
    return lax.axis_index("s") * NC + lax.axis_index("c")


def _hist_body(dst_hbm, out_hbm, dst_v, acc_v, sem):
    t = _wid()
    cp = pltpu.async_copy(dst_hbm.at[pl.ds(t * ET, ET)], dst_v, sem)

    @plsc.parallel_loop(0, NP // 256, unroll=4)
    def _(j):
        base = j * 256
        for u in range(16):
            acc_v[pl.ds(base + u * 16, 16)] = jnp.zeros((16,), jnp.float32)

    cp.wait()
    ones = jnp.ones((16,), jnp.float32)

    @plsc.parallel_loop(0, CH, unroll=16)
    def _(i):
        d = dst_v[pl.ds(i * 16, 16)]
        plsc.addupdate_scatter(acc_v, [d], ones)
    pltpu.sync_copy(acc_v, out_hbm.at[t])


_hist = pl.kernel(
    _hist_body,
    out_type=jax.ShapeDtypeStruct((NT, NP), jnp.float32),
    mesh=_MESH,
    compiler_params=_SC_PARAMS,
    scratch_types=[
        pltpu.VMEM((ET,), jnp.int32),
        pltpu.VMEM((NP,), jnp.float32),
        pltpu.SemaphoreType.DMA,
    ],
)


def _dense_body(x_ref, we_ref, be_ref, wl_ref, bl_ref, wc_ref, hc_ref):
    hi = lax.Precision.HIGHEST
    m1 = jnp.dot(we_ref[...], wl_ref[...], precision=hi)
    wf = jnp.dot(m1, wc_ref[...], precision=hi)
    b2 = jnp.dot(be_ref[...], wl_ref[...], precision=hi) + bl_ref[...]
    bft = lax.dot_general(wc_ref[...], b2, (((0,), (1,)), ((), ())),
                          precision=hi)
    hct = lax.dot_general(wf, x_ref[...], (((0,), (1,)), ((), ())),
                          precision=hi) + bft
    hc_ref[...] = jnp.concatenate(
        [hct, jnp.zeros((3, NP - N), jnp.float32)], axis=1)


_dense = pl.pallas_call(
    _dense_body,
    out_shape=jax.ShapeDtypeStruct((3, NP), jnp.float32),
)


def _scale_body(hc_ref, hp_ref, gp_ref):
    deg = jnp.sum(hp_ref[...], axis=0, keepdims=True) + 1.0
    dinv = lax.rsqrt(deg)
    g3 = hc_ref[...] * dinv
    gp_ref[...] = jnp.concatenate(
        [g3, dinv, jnp.zeros((4, NP), jnp.float32)], axis=0)


_scale = pl.pallas_call(
    _scale_body,
    out_shape=jax.ShapeDtypeStruct((8, NP), jnp.float32),
)


def _msg_body(src_hbm, dst_hbm, gp_hbm, out_hbm,
              src_v, dst_v, g0, g1, g2, a0, a1, a2, sem):
    t = _wid()
    cps = [
        pltpu.async_copy(src_hbm.at[pl.ds(t * ET, ET)], src_v, sem),
        pltpu.async_copy(dst_hbm.at[pl.ds(t * ET, ET)], dst_v, sem),
        pltpu.async_copy(gp_hbm.at[0], g0, sem),
        pltpu.async_copy(gp_hbm.at[1], g1, sem),
        pltpu.async_copy(gp_hbm.at[2], g2, sem),
    ]

    @plsc.parallel_loop(0, NP // 128, unroll=4)
    def _(j):
        base = j * 128
        for u in range(8):
            z = jnp.zeros((16,), jnp.float32)
            a0[pl.ds(base + u * 16, 16)] = z
            a1[pl.ds(base + u * 16, 16)] = z
            a2[pl.ds(base + u * 16, 16)] = z

    for cp in cps:
        cp.wait()

    @plsc.parallel_loop(0, CH, unroll=16)
    def _(i):
        s = src_v[pl.ds(i * 16, 16)]
        d = dst_v[pl.ds(i * 16, 16)]
        v0 = plsc.load_gather(g0, [s])
        v1 = plsc.load_gather(g1, [s])
        v2 = plsc.load_gather(g2, [s])
        plsc.addupdate_scatter(a0, [d], v0)
        plsc.addupdate_scatter(a1, [d], v1)
        plsc.addupdate_scatter(a2, [d], v2)
    pltpu.sync_copy(a0, out_hbm.at[t])
    pltpu.sync_copy(a1, out_hbm.at[NT + t])
    pltpu.sync_copy(a2, out_hbm.at[2 * NT + t])


_msg = pl.kernel(
    _msg_body,
    out_type=jax.ShapeDtypeStruct((3 * NT, NP), jnp.float32),
    mesh=_MESH,
    compiler_params=_SC_PARAMS,
    scratch_types=[
        pltpu.VMEM((ET,), jnp.int32),
        pltpu.VMEM((ET,), jnp.int32),
        pltpu.VMEM((NP,), jnp.float32),
        pltpu.VMEM((NP,), jnp.float32),
        pltpu.VMEM((NP,), jnp.float32),
        pltpu.VMEM((NP,), jnp.float32),
        pltpu.VMEM((NP,), jnp.float32),
        pltpu.VMEM((NP,), jnp.float32),
        pltpu.SemaphoreType.DMA,
    ],
)


def _final_body(sp_ref, gp_ref, bc_ref, out_ref):
    dinv = gp_ref[3:4, :]
    rows = []
    for c in range(3):
        s = jnp.sum(sp_ref[c * NT:(c + 1) * NT, :], axis=0, keepdims=True)
        rows.append(dinv * (s + gp_ref[c:c + 1, :]) + bc_ref[0:1, c:c + 1])
    out_ref[...] = jnp.concatenate(
        rows + [jnp.zeros((5, NP), jnp.float32)], axis=0)


_final = pl.pallas_call(
    _final_body,
    out_shape=jax.ShapeDtypeStruct((8, NP), jnp.float32),
)


def kernel(x, edge_index, W_emb, b_emb, W_lin, b_lin, W_conv, b_conv):
    src = edge_index[0]
    dst = edge_index[1]
    hp = _hist(dst)
    hc = _dense(x, W_emb, b_emb.reshape(1, -1), W_lin, b_lin.reshape(1, -1),
                W_conv)
    gp = _scale(hc, hp)
    sp = _msg(src, dst, gp)
    out_t = _final(sp, gp, b_conv.reshape(1, -1))
    return out_t[:3, :N].T
